# Initial kernel scaffold; baseline (speedup 1.0000x reference)
import jax
import jax.numpy as jnp
from jax.experimental import pallas as pl


def kernel(x, edge_index, edge_attr, We1, Ws1, Wr1, be1, Wn1, Wi1, bn1, We2, Ws2, Wr2, be2, Wn2, Wi2, bn2, Wgn, Wge, bg):
    raise NotImplementedError("write your pallas kernel here")


# speedup vs baseline: 2.3734x; 2.3734x over previous
import functools

import jax
import jax.numpy as jnp
from jax import lax
from jax.experimental import pallas as pl
from jax.experimental.pallas import tpu as pltpu
from jax.experimental.pallas import tpu_sc as plsc

N = 10000
E = 320000
EH = E // 2
DN = 128
DE = 16
H1 = 256
H2 = 128

NS = 16
NP = 10112
RPT = NP // NS
BLK = 128
NBLK_ALL = E // BLK
NBLK_HALF = EH // BLK
KMAX_ALL = -(-NBLK_ALL // NS)
KMAX_HALF = -(-NBLK_HALF // NS)


def _mesh():
    return plsc.VectorSubcoreMesh(
        core_axis_name="c", subcore_axis_name="s", num_cores=2, num_subcores=NS)


def _zero_rows(zbuf, n_vregs):
    def row(i, _):
        for j in range(n_vregs):
            zbuf[i, pl.ds(j * 16, 16)] = jnp.zeros((16,), jnp.float32)
        return 0
    lax.fori_loop(0, 128, row, 0)


def _zero_stripe(s, zbuf, acc):
    off = 0
    for nrows in (128, 128, 128, 128, 120):
        pltpu.sync_copy(zbuf.at[pl.ds(0, nrows)],
                        acc.at[pl.ds(s * RPT + off, nrows)])
        off += nrows


def _sc_counts(dst2):
    def body(dst2_hbm, cnt0, cnt1, idx_v, ones_v, acc):
        c = lax.axis_index("c")
        s = lax.axis_index("s")
        _zero_rows(ones_v, 8)
        _zero_stripe(s, ones_v, acc)
        lane = lax.iota(jnp.int32, 16)
        onerow = jnp.where(lane == 0, 1.0, 0.0)

        def orow(i, _):
            ones_v[i, pl.ds(0, 16)] = onerow
            return 0
        lax.fori_loop(0, 128, orow, 0)
        plsc.subcore_barrier()

        def sweep(cc):
            def blk(k, _):
                b = s + NS * k

                @pl.when(b < NBLK_HALF)
                def _():
                    pltpu.sync_copy(dst2_hbm.at[cc, pl.ds(b * BLK, BLK)], idx_v)
                    pltpu.sync_copy(ones_v, acc.at[idx_v], add=True)
                return 0
            lax.fori_loop(0, KMAX_HALF, blk, 0)

        @pl.when(c == 0)
        def _():
            sweep(0)

        @pl.when(c == 1)
        def _():
            sweep(1)
        plsc.subcore_barrier()

        stripe = acc.at[pl.ds(s * RPT, RPT)]

        @pl.when(c == 0)
        def _():
            pltpu.sync_copy(stripe, cnt0.at[pl.ds(s * RPT, RPT)])

        @pl.when(c == 1)
        def _():
            pltpu.sync_copy(stripe, cnt1.at[pl.ds(s * RPT, RPT)])

    f = pl.kernel(
        body,
        out_type=[jax.ShapeDtypeStruct((NP, 128), jnp.float32),
                  jax.ShapeDtypeStruct((NP, 128), jnp.float32)],
        mesh=_mesh(),
        scratch_types=[
            pltpu.VMEM((BLK,), jnp.int32),
            pltpu.VMEM((128, 128), jnp.float32),
            pltpu.VMEM_SHARED((NP, 128), jnp.float32),
        ],
    )
    return f(dst2)


def _sc_pass1(src_i, dst_i, tsa, tsb, tra, trb, ea2):
    def body(src_hbm, dst_hbm, tsa_h, tsb_h, tra_h, trb_h, ea_h,
             e1_out, agg_out, idx_s, idx_d, gs, gr, eab, acc):
        c = lax.axis_index("c")
        s = lax.axis_index("s")
        _zero_rows(gs, 8)
        _zero_stripe(s, gs, acc)
        plsc.subcore_barrier()

        def sweep(cc, ts_h, tr_h):
            def blk(k, _):
                b = s + NS * k

                @pl.when(b < NBLK_ALL)
                def _():
                    base = b * BLK
                    pltpu.sync_copy(src_hbm.at[pl.ds(base, BLK)], idx_s)
                    pltpu.sync_copy(dst_hbm.at[pl.ds(base, BLK)], idx_d)
                    pltpu.sync_copy(ts_h.at[idx_s], gs)
                    pltpu.sync_copy(tr_h.at[idx_d], gr)
                    pltpu.sync_copy(ea_h.at[cc, pl.ds(base, BLK)], eab)

                    def row(i, _):
                        for j in range(8):
                            sl = pl.ds(j * 16, 16)
                            v = gs[i, sl] + gr[i, sl] + eab[i, sl]
                            gs[i, sl] = jnp.maximum(v, 0.0)
                        return 0
                    lax.fori_loop(0, BLK, row, 0)
                    pltpu.sync_copy(gs, e1_out.at[cc, pl.ds(base, BLK)])
                    pltpu.sync_copy(gs, acc.at[idx_d], add=True)
                return 0
            lax.fori_loop(0, KMAX_ALL, blk, 0)

        @pl.when(c == 0)
        def _():
            sweep(0, tsa_h, tra_h)

        @pl.when(c == 1)
        def _():
            sweep(1, tsb_h, trb_h)
        plsc.subcore_barrier()

        stripe = acc.at[pl.ds(s * RPT, RPT)]

        @pl.when(c == 0)
        def _():
            pltpu.sync_copy(stripe, agg_out.at[0, pl.ds(s * RPT, RPT)])

        @pl.when(c == 1)
        def _():
            pltpu.sync_copy(stripe, agg_out.at[1, pl.ds(s * RPT, RPT)])

    f = pl.kernel(
        body,
        out_type=[jax.ShapeDtypeStruct((2, E, 128), jnp.float32),
                  jax.ShapeDtypeStruct((2, NP, 128), jnp.float32)],
        mesh=_mesh(),
        scratch_types=[
            pltpu.VMEM((BLK,), jnp.int32),
            pltpu.VMEM((BLK,), jnp.int32),
            pltpu.VMEM((BLK, 128), jnp.float32),
            pltpu.VMEM((BLK, 128), jnp.float32),
            pltpu.VMEM((BLK, 128), jnp.float32),
            pltpu.VMEM_SHARED((NP, 128), jnp.float32),
        ],
    )
    return f(src_i, dst_i, tsa, tsb, tra, trb, ea2)


def _sc_pass2(src2, dst2, ns2, nr2, ew2):
    def body(src_hbm, dst_hbm, ns_h, nr_h, ew_h,
             agg0, agg1, idx_s, idx_d, gs, gr, ewb, acc):
        c = lax.axis_index("c")
        s = lax.axis_index("s")
        _zero_rows(gs, 8)
        _zero_stripe(s, gs, acc)
        plsc.subcore_barrier()

        def sweep(cc):
            def blk(k, _):
                b = s + NS * k

                @pl.when(b < NBLK_HALF)
                def _():
                    base = b * BLK
                    pltpu.sync_copy(src_hbm.at[cc, pl.ds(base, BLK)], idx_s)
                    pltpu.sync_copy(dst_hbm.at[cc, pl.ds(base, BLK)], idx_d)
                    pltpu.sync_copy(ns_h.at[idx_s], gs)
                    pltpu.sync_copy(nr_h.at[idx_d], gr)
                    pltpu.sync_copy(ew_h.at[cc, pl.ds(base, BLK)], ewb)

                    def row(i, _):
                        for j in range(8):
                            sl = pl.ds(j * 16, 16)
                            v = gs[i, sl] + gr[i, sl] + ewb[i, sl]
                            gs[i, sl] = jnp.maximum(v, 0.0)
                        return 0
                    lax.fori_loop(0, BLK, row, 0)
                    pltpu.sync_copy(gs, acc.at[idx_d], add=True)
                return 0
            lax.fori_loop(0, KMAX_HALF, blk, 0)

        @pl.when(c == 0)
        def _():
            sweep(0)

        @pl.when(c == 1)
        def _():
            sweep(1)
        plsc.subcore_barrier()

        stripe = acc.at[pl.ds(s * RPT, RPT)]

        @pl.when(c == 0)
        def _():
            pltpu.sync_copy(stripe, agg0.at[pl.ds(s * RPT, RPT)])

        @pl.when(c == 1)
        def _():
            pltpu.sync_copy(stripe, agg1.at[pl.ds(s * RPT, RPT)])

    f = pl.kernel(
        body,
        out_type=[jax.ShapeDtypeStruct((NP, 128), jnp.float32),
                  jax.ShapeDtypeStruct((NP, 128), jnp.float32)],
        mesh=_mesh(),
        scratch_types=[
            pltpu.VMEM((BLK,), jnp.int32),
            pltpu.VMEM((BLK,), jnp.int32),
            pltpu.VMEM((BLK, 128), jnp.float32),
            pltpu.VMEM((BLK, 128), jnp.float32),
            pltpu.VMEM((BLK, 128), jnp.float32),
            pltpu.VMEM_SHARED((NP, 128), jnp.float32),
        ],
    )
    return f(src2, dst2, ns2, nr2, ew2)


def _tc_node_proj(x, wcat):
    BN = 1000

    def body(xr, wr, outr):
        outr[0] = jnp.dot(xr[...], wr[...], preferred_element_type=jnp.float32)

    return pl.pallas_call(
        body,
        grid=(N // BN, 6),
        in_specs=[pl.BlockSpec((BN, 128), lambda i, j: (i, 0)),
                  pl.BlockSpec((128, 128), lambda i, j: (0, j))],
        out_specs=pl.BlockSpec((1, BN, 128), lambda i, j: (j, i, 0)),
        out_shape=jax.ShapeDtypeStruct((6, N, 128), jnp.float32),
    )(x, wcat)


def _tc_edge_proj(edge_attr, we1, be1r):
    BE = 2000

    def body(ear, wr, br, outr):
        outr[0] = (jnp.dot(ear[...], wr[...], preferred_element_type=jnp.float32)
                   + br[0, 0:1])

    return pl.pallas_call(
        body,
        grid=(E // BE, 2),
        in_specs=[pl.BlockSpec((BE, 16), lambda i, j: (i, 0)),
                  pl.BlockSpec((16, 128), lambda i, j: (0, j)),
                  pl.BlockSpec((1, 8, 128), lambda i, j: (j, 0, 0))],
        out_specs=pl.BlockSpec((1, BE, 128), lambda i, j: (j, i, 0)),
        out_shape=jax.ShapeDtypeStruct((2, E, 128), jnp.float32),
    )(edge_attr, we1, be1r)


def _tc_node1(agg1h, cnt0, cnt1, p1, wi1, w2cat, bn1r, b3r):
    BN = 1000

    def body(aggr, c0r, c1r, p1r, wi1r, w2r, b1r, b3rr, outr):
        cnt = c0r[...][:, 0:1] + c1r[...][:, 0:1]
        dinv = 1.0 / jnp.maximum(cnt, 1.0)
        agg1 = jnp.concatenate([aggr[0], aggr[1]], axis=1) * dinv
        xn = jnp.concatenate([p1r[0], p1r[1]], axis=1)
        n1 = jnp.maximum(
            jnp.dot(agg1, wi1r[...], preferred_element_type=jnp.float32)
            + xn + b1r[...], 0.0)
        outr[...] = (jnp.dot(n1, w2r[...], preferred_element_type=jnp.float32)
                     + b3rr[...])

    return pl.pallas_call(
        body,
        grid=(N // BN,),
        in_specs=[pl.BlockSpec((2, BN, 128), lambda i: (0, i, 0)),
                  pl.BlockSpec((BN, 128), lambda i: (i, 0)),
                  pl.BlockSpec((BN, 128), lambda i: (i, 0)),
                  pl.BlockSpec((2, BN, 128), lambda i: (2, i, 0)),
                  pl.BlockSpec((256, 256), lambda i: (0, 0)),
                  pl.BlockSpec((256, 384), lambda i: (0, 0)),
                  pl.BlockSpec((1, 256), lambda i: (0, 0)),
                  pl.BlockSpec((1, 384), lambda i: (0, 0))],
        out_specs=pl.BlockSpec((BN, 384), lambda i: (i, 0)),
        out_shape=jax.ShapeDtypeStruct((N, 384), jnp.float32),
    )(agg1h, cnt0, cnt1, p1, wi1, w2cat, bn1r, b3r)


def _tc_edge2_mm(e1h, we2r, be2r):
    BE = 2000

    def body(er, wr, br, outr):
        outr[...] = (jnp.dot(er[0], wr[0], preferred_element_type=jnp.float32)
                     + jnp.dot(er[1], wr[1], preferred_element_type=jnp.float32)
                     + br[...])

    return pl.pallas_call(
        body,
        grid=(E // BE,),
        in_specs=[pl.BlockSpec((2, BE, 128), lambda i: (0, i, 0)),
                  pl.BlockSpec((2, 128, 128), lambda i: (0, 0, 0)),
                  pl.BlockSpec((1, 128), lambda i: (0, 0))],
        out_specs=pl.BlockSpec((BE, 128), lambda i: (i, 0)),
        out_shape=jax.ShapeDtypeStruct((E, 128), jnp.float32),
    )(e1h, we2r, be2r)


def _tc_final_partials(o2, agg2p0, agg2p1, cnt0, cnt1, wi2):
    BN = 1000
    NSTEP = N // BN

    def body(y2r, p0r, p1r, c0r, c1r, wi2r, outn, oute):
        agg2u = p0r[...] + p1r[...]
        oute[0] = jnp.broadcast_to(
            jnp.sum(agg2u, axis=0, keepdims=True), (8, 128))
        cnt = c0r[...][:, 0:1] + c1r[...][:, 0:1]
        dinv = 1.0 / jnp.maximum(cnt, 1.0)
        n2 = jnp.maximum(
            y2r[...] + jnp.dot(agg2u * dinv, wi2r[...],
                               preferred_element_type=jnp.float32), 0.0)
        outn[0] = jnp.broadcast_to(jnp.sum(n2, axis=0, keepdims=True), (8, 128))

    return pl.pallas_call(
        body,
        grid=(NSTEP,),
        in_specs=[pl.BlockSpec((BN, 128), lambda i: (i, 2)),
                  pl.BlockSpec((BN, 128), lambda i: (i, 0)),
                  pl.BlockSpec((BN, 128), lambda i: (i, 0)),
                  pl.BlockSpec((BN, 128), lambda i: (i, 0)),
                  pl.BlockSpec((BN, 128), lambda i: (i, 0)),
                  pl.BlockSpec((128, 128), lambda i: (0, 0))],
        out_specs=[pl.BlockSpec((1, 8, 128), lambda i: (i, 0, 0)),
                   pl.BlockSpec((1, 8, 128), lambda i: (i, 0, 0))],
        out_shape=[jax.ShapeDtypeStruct((NSTEP, 8, 128), jnp.float32),
                   jax.ShapeDtypeStruct((NSTEP, 8, 128), jnp.float32)],
    )(o2, agg2p0, agg2p1, cnt0, cnt1, wi2)


def _tc_readout(pn, pe, wgp):
    def body(pnr, per, wgpr, outr):
        sn = jnp.sum(pnr[...][:, 0, :], axis=0, keepdims=True)
        se = jnp.sum(per[...][:, 0, :], axis=0, keepdims=True)
        v = jnp.concatenate([sn / float(N), se / float(E)], axis=1)
        outr[...] = jnp.broadcast_to(
            jnp.dot(v, wgpr[...], preferred_element_type=jnp.float32), (8, 128))

    nstep = pn.shape[0]
    return pl.pallas_call(
        body,
        in_specs=[pl.BlockSpec((nstep, 8, 128), lambda: (0, 0, 0)),
                  pl.BlockSpec((nstep, 8, 128), lambda: (0, 0, 0)),
                  pl.BlockSpec((256, 128), lambda: (0, 0))],
        out_specs=pl.BlockSpec((8, 128), lambda: (0, 0)),
        out_shape=jax.ShapeDtypeStruct((8, 128), jnp.float32),
    )(pn, pe, wgp)


def kernel(x, edge_index, edge_attr, We1, Ws1, Wr1, be1, Wn1, Wi1, bn1,
           We2, Ws2, Wr2, be2, Wn2, Wi2, bn2, Wgn, Wge, bg):
    ei = edge_index.astype(jnp.int32)
    src = ei[0]
    dst = ei[1]
    src2 = src.reshape(2, EH)
    dst2 = dst.reshape(2, EH)

    cnt0, cnt1 = _sc_counts(dst2)

    w1cat = jnp.concatenate([Ws1, Wr1, Wn1], axis=1)
    p1 = _tc_node_proj(x, w1cat)
    tsa, tsb, tra, trb = p1[0], p1[1], p1[2], p1[3]
    be1r = jnp.broadcast_to(be1.reshape(2, 1, 128), (2, 8, 128))
    ea2 = _tc_edge_proj(edge_attr, We1, be1r)

    e1h, agg1h = _sc_pass1(src, dst, tsa, tsb, tra, trb, ea2)

    w2cat = jnp.concatenate([Ws2, Wr2, Wn2], axis=1)
    b3r = jnp.concatenate([jnp.zeros((256,), jnp.float32), bn2]).reshape(1, 384)
    o2 = _tc_node1(agg1h, cnt0, cnt1, p1, Wi1, w2cat,
                   bn1.reshape(1, 256), b3r)
    ns2 = o2[:, 0:128]
    nr2 = o2[:, 128:256]

    ew = _tc_edge2_mm(e1h, We2.reshape(2, 128, 128), be2.reshape(1, 128))
    ew2 = ew.reshape(2, EH, 128)

    agg2p0, agg2p1 = _sc_pass2(src2, dst2, ns2, nr2, ew2)

    wgp = jnp.pad(jnp.concatenate([Wgn, Wge], axis=0), ((0, 0), (0, 127)))
    pn, pe = _tc_final_partials(o2, agg2p0, agg2p1, cnt0, cnt1, Wi2)
    out = _tc_readout(pn, pe, wgp)
    return out[0, 0:1] + bg


# speedup vs baseline: 2.4681x; 1.0399x over previous
import functools

import jax
import jax.numpy as jnp
from jax import lax
from jax.experimental import pallas as pl
from jax.experimental.pallas import tpu as pltpu
from jax.experimental.pallas import tpu_sc as plsc

N = 10000
E = 320000
EH = E // 2
DN = 128
DE = 16
H1 = 256
H2 = 128

NS = 16
NP = 10112
RPT = NP // NS
BLK = 128
NBLK_ALL = E // BLK
NBLK_HALF = EH // BLK
KMAX_ALL = -(-NBLK_ALL // NS)
KMAX_HALF = -(-NBLK_HALF // NS)


def _mesh():
    return plsc.VectorSubcoreMesh(
        core_axis_name="c", subcore_axis_name="s", num_cores=2, num_subcores=NS)


def _zero_rows(zbuf, n_vregs):
    def row(i, _):
        for j in range(n_vregs):
            zbuf[i, pl.ds(j * 16, 16)] = jnp.zeros((16,), jnp.float32)
        return 0
    lax.fori_loop(0, 128, row, 0)


def _zero_stripe(s, zbuf, acc):
    off = 0
    for nrows in (128, 128, 128, 128, 120):
        pltpu.sync_copy(zbuf.at[pl.ds(0, nrows)],
                        acc.at[pl.ds(s * RPT + off, nrows)])
        off += nrows


def _sc_counts(dst2):
    def body(dst2_hbm, cnt0, cnt1, idx_v, ones_v, acc):
        c = lax.axis_index("c")
        s = lax.axis_index("s")
        _zero_rows(ones_v, 8)
        _zero_stripe(s, ones_v, acc)
        lane = lax.iota(jnp.int32, 16)
        onerow = jnp.where(lane == 0, 1.0, 0.0)

        def orow(i, _):
            ones_v[i, pl.ds(0, 16)] = onerow
            return 0
        lax.fori_loop(0, 128, orow, 0)
        plsc.subcore_barrier()

        def sweep(cc):
            def blk(k, _):
                b = s + NS * k

                @pl.when(b < NBLK_HALF)
                def _():
                    pltpu.sync_copy(dst2_hbm.at[cc, pl.ds(b * BLK, BLK)], idx_v)
                    pltpu.sync_copy(ones_v, acc.at[idx_v], add=True)
                return 0
            lax.fori_loop(0, KMAX_HALF, blk, 0)

        @pl.when(c == 0)
        def _():
            sweep(0)

        @pl.when(c == 1)
        def _():
            sweep(1)
        plsc.subcore_barrier()

        stripe = acc.at[pl.ds(s * RPT, RPT)]

        @pl.when(c == 0)
        def _():
            pltpu.sync_copy(stripe, cnt0.at[pl.ds(s * RPT, RPT)])

        @pl.when(c == 1)
        def _():
            pltpu.sync_copy(stripe, cnt1.at[pl.ds(s * RPT, RPT)])

    f = pl.kernel(
        body,
        out_type=[jax.ShapeDtypeStruct((NP, 128), jnp.float32),
                  jax.ShapeDtypeStruct((NP, 128), jnp.float32)],
        mesh=_mesh(),
        scratch_types=[
            pltpu.VMEM((BLK,), jnp.int32),
            pltpu.VMEM((128, 128), jnp.float32),
            pltpu.VMEM_SHARED((NP, 128), jnp.float32),
        ],
    )
    return f(dst2)


def _sc_pass1(ei, tsa, tsb, tra, trb, ea2):
    def body(ei_h, tsa_h, tsb_h, tra_h, trb_h, ea_h,
             e1_out, agg_out, idx2, gs, gr, eab, acc):
        c = lax.axis_index("c")
        s = lax.axis_index("s")
        _zero_rows(gs, 8)
        _zero_stripe(s, gs, acc)
        plsc.subcore_barrier()

        def sweep(cc, ts_h, tr_h):
            def blk(k, _):
                b = s + NS * k

                @pl.when(b < NBLK_ALL)
                def _():
                    base = b * BLK
                    pltpu.sync_copy(ei_h.at[:, pl.ds(base, BLK)], idx2)
                    pltpu.sync_copy(ts_h.at[idx2.at[0]], gs)
                    pltpu.sync_copy(tr_h.at[idx2.at[1]], gr)
                    pltpu.sync_copy(ea_h.at[cc, pl.ds(base, BLK)], eab)

                    @plsc.parallel_loop(0, BLK, step=1, unroll=4)
                    def row(i):
                        for j in range(8):
                            sl = pl.ds(j * 16, 16)
                            v = gs[i, sl] + gr[i, sl] + eab[i, sl]
                            gs[i, sl] = jnp.maximum(v, 0.0)
                    pltpu.sync_copy(gs, e1_out.at[cc, pl.ds(base, BLK)])
                    pltpu.sync_copy(gs, acc.at[idx2.at[1]], add=True)
                return 0
            lax.fori_loop(0, KMAX_ALL, blk, 0)

        @pl.when(c == 0)
        def _():
            sweep(0, tsa_h, tra_h)

        @pl.when(c == 1)
        def _():
            sweep(1, tsb_h, trb_h)
        plsc.subcore_barrier()

        stripe = acc.at[pl.ds(s * RPT, RPT)]

        @pl.when(c == 0)
        def _():
            pltpu.sync_copy(stripe, agg_out.at[0, pl.ds(s * RPT, RPT)])

        @pl.when(c == 1)
        def _():
            pltpu.sync_copy(stripe, agg_out.at[1, pl.ds(s * RPT, RPT)])

    f = pl.kernel(
        body,
        out_type=[jax.ShapeDtypeStruct((2, E, 128), jnp.float32),
                  jax.ShapeDtypeStruct((2, NP, 128), jnp.float32)],
        mesh=_mesh(),
        scratch_types=[
            pltpu.VMEM((2, BLK), jnp.int32),
            pltpu.VMEM((BLK, 128), jnp.float32),
            pltpu.VMEM((BLK, 128), jnp.float32),
            pltpu.VMEM((BLK, 128), jnp.float32),
            pltpu.VMEM_SHARED((NP, 128), jnp.float32),
        ],
    )
    return f(ei, tsa, tsb, tra, trb, ea2)


def _sc_pass2(sdh, ns2, nr2, ew2):
    def body(sd_h, ns_h, nr_h, ew_h,
             agg0, agg1, idx2, gs, gr, ewb, acc):
        c = lax.axis_index("c")
        s = lax.axis_index("s")
        _zero_rows(gs, 8)
        _zero_stripe(s, gs, acc)
        plsc.subcore_barrier()

        def sweep(cc):
            def blk(k, _):
                b = s + NS * k

                @pl.when(b < NBLK_HALF)
                def _():
                    base = b * BLK
                    pltpu.sync_copy(sd_h.at[cc, :, pl.ds(base, BLK)], idx2)
                    pltpu.sync_copy(ns_h.at[idx2.at[0]], gs)
                    pltpu.sync_copy(nr_h.at[idx2.at[1]], gr)
                    pltpu.sync_copy(ew_h.at[cc, pl.ds(base, BLK)], ewb)

                    @plsc.parallel_loop(0, BLK, step=1, unroll=4)
                    def row(i):
                        for j in range(8):
                            sl = pl.ds(j * 16, 16)
                            v = gs[i, sl] + gr[i, sl] + ewb[i, sl]
                            gs[i, sl] = jnp.maximum(v, 0.0)
                    pltpu.sync_copy(gs, acc.at[idx2.at[1]], add=True)
                return 0
            lax.fori_loop(0, KMAX_HALF, blk, 0)

        @pl.when(c == 0)
        def _():
            sweep(0)

        @pl.when(c == 1)
        def _():
            sweep(1)
        plsc.subcore_barrier()

        stripe = acc.at[pl.ds(s * RPT, RPT)]

        @pl.when(c == 0)
        def _():
            pltpu.sync_copy(stripe, agg0.at[pl.ds(s * RPT, RPT)])

        @pl.when(c == 1)
        def _():
            pltpu.sync_copy(stripe, agg1.at[pl.ds(s * RPT, RPT)])

    f = pl.kernel(
        body,
        out_type=[jax.ShapeDtypeStruct((NP, 128), jnp.float32),
                  jax.ShapeDtypeStruct((NP, 128), jnp.float32)],
        mesh=_mesh(),
        scratch_types=[
            pltpu.VMEM((2, BLK), jnp.int32),
            pltpu.VMEM((BLK, 128), jnp.float32),
            pltpu.VMEM((BLK, 128), jnp.float32),
            pltpu.VMEM((BLK, 128), jnp.float32),
            pltpu.VMEM_SHARED((NP, 128), jnp.float32),
        ],
    )
    return f(sdh, ns2, nr2, ew2)


def _tc_node_proj(x, wcat):
    BN = 1000

    def body(xr, wr, outr):
        outr[0] = jnp.dot(xr[...], wr[...], preferred_element_type=jnp.float32)

    return pl.pallas_call(
        body,
        grid=(N // BN, 6),
        in_specs=[pl.BlockSpec((BN, 128), lambda i, j: (i, 0)),
                  pl.BlockSpec((128, 128), lambda i, j: (0, j))],
        out_specs=pl.BlockSpec((1, BN, 128), lambda i, j: (j, i, 0)),
        out_shape=jax.ShapeDtypeStruct((6, N, 128), jnp.float32),
    )(x, wcat)


def _tc_edge_proj(edge_attr, we1, be1r):
    BE = 2000

    def body(ear, wr, br, outr):
        outr[0] = (jnp.dot(ear[...], wr[...], preferred_element_type=jnp.float32)
                   + br[0, 0:1])

    return pl.pallas_call(
        body,
        grid=(E // BE, 2),
        in_specs=[pl.BlockSpec((BE, 16), lambda i, j: (i, 0)),
                  pl.BlockSpec((16, 128), lambda i, j: (0, j)),
                  pl.BlockSpec((1, 8, 128), lambda i, j: (j, 0, 0))],
        out_specs=pl.BlockSpec((1, BE, 128), lambda i, j: (j, i, 0)),
        out_shape=jax.ShapeDtypeStruct((2, E, 128), jnp.float32),
    )(edge_attr, we1, be1r)


def _tc_node1(agg1h, cnt0, cnt1, p1, wi1, w2cat, bn1r, b3r):
    BN = 1000

    def body(aggr, c0r, c1r, p1r, wi1r, w2r, b1r, b3rr, outr):
        cnt = c0r[...][:, 0:1] + c1r[...][:, 0:1]
        dinv = 1.0 / jnp.maximum(cnt, 1.0)
        agg1 = jnp.concatenate([aggr[0], aggr[1]], axis=1) * dinv
        xn = jnp.concatenate([p1r[0], p1r[1]], axis=1)
        n1 = jnp.maximum(
            jnp.dot(agg1, wi1r[...], preferred_element_type=jnp.float32)
            + xn + b1r[...], 0.0)
        outr[...] = (jnp.dot(n1, w2r[...], preferred_element_type=jnp.float32)
                     + b3rr[...])

    return pl.pallas_call(
        body,
        grid=(N // BN,),
        in_specs=[pl.BlockSpec((2, BN, 128), lambda i: (0, i, 0)),
                  pl.BlockSpec((BN, 128), lambda i: (i, 0)),
                  pl.BlockSpec((BN, 128), lambda i: (i, 0)),
                  pl.BlockSpec((2, BN, 128), lambda i: (2, i, 0)),
                  pl.BlockSpec((256, 256), lambda i: (0, 0)),
                  pl.BlockSpec((256, 384), lambda i: (0, 0)),
                  pl.BlockSpec((1, 256), lambda i: (0, 0)),
                  pl.BlockSpec((1, 384), lambda i: (0, 0))],
        out_specs=pl.BlockSpec((BN, 384), lambda i: (i, 0)),
        out_shape=jax.ShapeDtypeStruct((N, 384), jnp.float32),
    )(agg1h, cnt0, cnt1, p1, wi1, w2cat, bn1r, b3r)


def _tc_edge2_mm(e1h, we2r, be2r):
    BE = 2000

    def body(er, wr, br, outr):
        outr[...] = (jnp.dot(er[0], wr[0], preferred_element_type=jnp.float32)
                     + jnp.dot(er[1], wr[1], preferred_element_type=jnp.float32)
                     + br[...])

    return pl.pallas_call(
        body,
        grid=(E // BE,),
        in_specs=[pl.BlockSpec((2, BE, 128), lambda i: (0, i, 0)),
                  pl.BlockSpec((2, 128, 128), lambda i: (0, 0, 0)),
                  pl.BlockSpec((1, 128), lambda i: (0, 0))],
        out_specs=pl.BlockSpec((BE, 128), lambda i: (i, 0)),
        out_shape=jax.ShapeDtypeStruct((E, 128), jnp.float32),
    )(e1h, we2r, be2r)


def _tc_final_partials(o2, agg2p0, agg2p1, cnt0, cnt1, wi2):
    BN = 1000
    NSTEP = N // BN

    def body(y2r, p0r, p1r, c0r, c1r, wi2r, outn, oute):
        agg2u = p0r[...] + p1r[...]
        oute[0] = jnp.broadcast_to(
            jnp.sum(agg2u, axis=0, keepdims=True), (8, 128))
        cnt = c0r[...][:, 0:1] + c1r[...][:, 0:1]
        dinv = 1.0 / jnp.maximum(cnt, 1.0)
        n2 = jnp.maximum(
            y2r[...] + jnp.dot(agg2u * dinv, wi2r[...],
                               preferred_element_type=jnp.float32), 0.0)
        outn[0] = jnp.broadcast_to(jnp.sum(n2, axis=0, keepdims=True), (8, 128))

    return pl.pallas_call(
        body,
        grid=(NSTEP,),
        in_specs=[pl.BlockSpec((BN, 128), lambda i: (i, 2)),
                  pl.BlockSpec((BN, 128), lambda i: (i, 0)),
                  pl.BlockSpec((BN, 128), lambda i: (i, 0)),
                  pl.BlockSpec((BN, 128), lambda i: (i, 0)),
                  pl.BlockSpec((BN, 128), lambda i: (i, 0)),
                  pl.BlockSpec((128, 128), lambda i: (0, 0))],
        out_specs=[pl.BlockSpec((1, 8, 128), lambda i: (i, 0, 0)),
                   pl.BlockSpec((1, 8, 128), lambda i: (i, 0, 0))],
        out_shape=[jax.ShapeDtypeStruct((NSTEP, 8, 128), jnp.float32),
                   jax.ShapeDtypeStruct((NSTEP, 8, 128), jnp.float32)],
    )(o2, agg2p0, agg2p1, cnt0, cnt1, wi2)


def _tc_readout(pn, pe, wgp):
    def body(pnr, per, wgpr, outr):
        sn = jnp.sum(pnr[...][:, 0, :], axis=0, keepdims=True)
        se = jnp.sum(per[...][:, 0, :], axis=0, keepdims=True)
        v = jnp.concatenate([sn / float(N), se / float(E)], axis=1)
        outr[...] = jnp.broadcast_to(
            jnp.dot(v, wgpr[...], preferred_element_type=jnp.float32), (8, 128))

    nstep = pn.shape[0]
    return pl.pallas_call(
        body,
        in_specs=[pl.BlockSpec((nstep, 8, 128), lambda: (0, 0, 0)),
                  pl.BlockSpec((nstep, 8, 128), lambda: (0, 0, 0)),
                  pl.BlockSpec((256, 128), lambda: (0, 0))],
        out_specs=pl.BlockSpec((8, 128), lambda: (0, 0)),
        out_shape=jax.ShapeDtypeStruct((8, 128), jnp.float32),
    )(pn, pe, wgp)


def kernel(x, edge_index, edge_attr, We1, Ws1, Wr1, be1, Wn1, Wi1, bn1,
           We2, Ws2, Wr2, be2, Wn2, Wi2, bn2, Wgn, Wge, bg):
    ei = edge_index.astype(jnp.int32)
    src = ei[0]
    dst = ei[1]
    src2 = src.reshape(2, EH)
    dst2 = dst.reshape(2, EH)

    cnt0, cnt1 = _sc_counts(dst2)

    w1cat = jnp.concatenate([Ws1, Wr1, Wn1], axis=1)
    p1 = _tc_node_proj(x, w1cat)
    tsa, tsb, tra, trb = p1[0], p1[1], p1[2], p1[3]
    be1r = jnp.broadcast_to(be1.reshape(2, 1, 128), (2, 8, 128))
    ea2 = _tc_edge_proj(edge_attr, We1, be1r)

    e1h, agg1h = _sc_pass1(ei, tsa, tsb, tra, trb, ea2)

    w2cat = jnp.concatenate([Ws2, Wr2, Wn2], axis=1)
    b3r = jnp.concatenate([jnp.zeros((256,), jnp.float32), bn2]).reshape(1, 384)
    o2 = _tc_node1(agg1h, cnt0, cnt1, p1, Wi1, w2cat,
                   bn1.reshape(1, 256), b3r)
    ns2 = o2[:, 0:128]
    nr2 = o2[:, 128:256]

    ew = _tc_edge2_mm(e1h, We2.reshape(2, 128, 128), be2.reshape(1, 128))
    ew2 = ew.reshape(2, EH, 128)

    sdh = jnp.stack([src2, dst2], axis=1)
    agg2p0, agg2p1 = _sc_pass2(sdh, ns2, nr2, ew2)

    wgp = jnp.pad(jnp.concatenate([Wgn, Wge], axis=0), ((0, 0), (0, 127)))
    pn, pe = _tc_final_partials(o2, agg2p0, agg2p1, cnt0, cnt1, Wi2)
    out = _tc_readout(pn, pe, wgp)
    return out[0, 0:1] + bg


# speedup vs baseline: 2.4724x; 1.0018x over previous
import functools

import jax
import jax.numpy as jnp
from jax import lax
from jax.experimental import pallas as pl
from jax.experimental.pallas import tpu as pltpu
from jax.experimental.pallas import tpu_sc as plsc

N = 10000
E = 320000
EH = E // 2
DN = 128
DE = 16
H1 = 256
H2 = 128

NS = 16
NP = 10112
RPT = NP // NS
BLK = 128
NBLK_ALL = E // BLK
NBLK_HALF = EH // BLK
KMAX_ALL = -(-NBLK_ALL // NS)
KMAX_HALF = -(-NBLK_HALF // NS)


def _mesh():
    return plsc.VectorSubcoreMesh(
        core_axis_name="c", subcore_axis_name="s", num_cores=2, num_subcores=NS)


def _zero_rows(zbuf, n_vregs):
    def row(i, _):
        for j in range(n_vregs):
            zbuf[i, pl.ds(j * 16, 16)] = jnp.zeros((16,), jnp.float32)
        return 0
    lax.fori_loop(0, 128, row, 0)


def _zero_stripe(s, zbuf, acc):
    off = 0
    for nrows in (128, 128, 128, 128, 120):
        pltpu.sync_copy(zbuf.at[pl.ds(0, nrows)],
                        acc.at[pl.ds(s * RPT + off, nrows)])
        off += nrows


def _sc_counts(dst2):
    def body(dst2_hbm, cnt0, cnt1, idx_v, ones_v, acc):
        c = lax.axis_index("c")
        s = lax.axis_index("s")
        _zero_rows(ones_v, 8)
        _zero_stripe(s, ones_v, acc)
        lane = lax.iota(jnp.int32, 16)
        onerow = jnp.where(lane == 0, 1.0, 0.0)

        def orow(i, _):
            ones_v[i, pl.ds(0, 16)] = onerow
            return 0
        lax.fori_loop(0, 128, orow, 0)
        plsc.subcore_barrier()

        def sweep(cc):
            def blk(k, _):
                b = s + NS * k

                @pl.when(b < NBLK_HALF)
                def _():
                    pltpu.sync_copy(dst2_hbm.at[cc, pl.ds(b * BLK, BLK)], idx_v)
                    pltpu.sync_copy(ones_v, acc.at[idx_v], add=True)
                return 0
            lax.fori_loop(0, KMAX_HALF, blk, 0)

        @pl.when(c == 0)
        def _():
            sweep(0)

        @pl.when(c == 1)
        def _():
            sweep(1)
        plsc.subcore_barrier()

        stripe = acc.at[pl.ds(s * RPT, RPT)]

        @pl.when(c == 0)
        def _():
            pltpu.sync_copy(stripe, cnt0.at[pl.ds(s * RPT, RPT)])

        @pl.when(c == 1)
        def _():
            pltpu.sync_copy(stripe, cnt1.at[pl.ds(s * RPT, RPT)])

    f = pl.kernel(
        body,
        out_type=[jax.ShapeDtypeStruct((NP, 128), jnp.float32),
                  jax.ShapeDtypeStruct((NP, 128), jnp.float32)],
        mesh=_mesh(),
        scratch_types=[
            pltpu.VMEM((BLK,), jnp.int32),
            pltpu.VMEM((128, 128), jnp.float32),
            pltpu.VMEM_SHARED((NP, 128), jnp.float32),
        ],
    )
    return f(dst2)


def _sc_pass1(ei, tsa, tsb, tra, trb, ea2):
    def body(ei_h, tsa_h, tsb_h, tra_h, trb_h, ea_h,
             e1_out, agg_out, idx2, gs, gr, eab, acc):
        c = lax.axis_index("c")
        s = lax.axis_index("s")
        _zero_rows(gs, 8)
        _zero_stripe(s, gs, acc)
        plsc.subcore_barrier()

        def sweep(cc, ts_h, tr_h):
            def blk(k, _):
                b = s + NS * k

                @pl.when(b < NBLK_ALL)
                def _():
                    base = b * BLK
                    pltpu.sync_copy(ei_h.at[:, pl.ds(base, BLK)], idx2)
                    pltpu.sync_copy(ts_h.at[idx2.at[0]], gs)
                    pltpu.sync_copy(tr_h.at[idx2.at[1]], gr)
                    pltpu.sync_copy(ea_h.at[cc, pl.ds(base, BLK)], eab)

                    @plsc.parallel_loop(0, BLK, step=1, unroll=4)
                    def row(i):
                        for j in range(8):
                            sl = pl.ds(j * 16, 16)
                            v = gs[i, sl] + gr[i, sl] + eab[i, sl]
                            gs[i, sl] = jnp.maximum(v, 0.0)
                    pltpu.sync_copy(gs, e1_out.at[cc, pl.ds(base, BLK)])
                    pltpu.sync_copy(gs, acc.at[idx2.at[1]], add=True)
                return 0
            lax.fori_loop(0, KMAX_ALL, blk, 0)

        @pl.when(c == 0)
        def _():
            sweep(0, tsa_h, tra_h)

        @pl.when(c == 1)
        def _():
            sweep(1, tsb_h, trb_h)
        plsc.subcore_barrier()

        stripe = acc.at[pl.ds(s * RPT, RPT)]

        @pl.when(c == 0)
        def _():
            pltpu.sync_copy(stripe, agg_out.at[0, pl.ds(s * RPT, RPT)])

        @pl.when(c == 1)
        def _():
            pltpu.sync_copy(stripe, agg_out.at[1, pl.ds(s * RPT, RPT)])

    f = pl.kernel(
        body,
        out_type=[jax.ShapeDtypeStruct((2, E, 128), jnp.float32),
                  jax.ShapeDtypeStruct((2, NP, 128), jnp.float32)],
        mesh=_mesh(),
        scratch_types=[
            pltpu.VMEM((2, BLK), jnp.int32),
            pltpu.VMEM((BLK, 128), jnp.float32),
            pltpu.VMEM((BLK, 128), jnp.float32),
            pltpu.VMEM((BLK, 128), jnp.float32),
            pltpu.VMEM_SHARED((NP, 128), jnp.float32),
        ],
    )
    return f(ei, tsa, tsb, tra, trb, ea2)


def _sc_pass2(sdh, ns2, nr2, ew2):
    def body(sd_h, ns_h, nr_h, ew_h,
             agg0, agg1, idx2, gs, gr, ewb, acc):
        c = lax.axis_index("c")
        s = lax.axis_index("s")
        _zero_rows(gs, 8)
        _zero_stripe(s, gs, acc)
        plsc.subcore_barrier()

        def sweep(cc):
            def blk(k, _):
                b = s + NS * k

                @pl.when(b < NBLK_HALF)
                def _():
                    base = b * BLK
                    pltpu.sync_copy(sd_h.at[cc, :, pl.ds(base, BLK)], idx2)
                    pltpu.sync_copy(ns_h.at[idx2.at[0]], gs)
                    pltpu.sync_copy(nr_h.at[idx2.at[1]], gr)
                    pltpu.sync_copy(ew_h.at[cc, pl.ds(base, BLK)], ewb)

                    @plsc.parallel_loop(0, BLK, step=1, unroll=4)
                    def row(i):
                        for j in range(8):
                            sl = pl.ds(j * 16, 16)
                            v = gs[i, sl] + gr[i, sl] + ewb[i, sl]
                            gs[i, sl] = jnp.maximum(v, 0.0)
                    pltpu.sync_copy(gs, acc.at[idx2.at[1]], add=True)
                return 0
            lax.fori_loop(0, KMAX_HALF, blk, 0)

        @pl.when(c == 0)
        def _():
            sweep(0)

        @pl.when(c == 1)
        def _():
            sweep(1)
        plsc.subcore_barrier()

        stripe = acc.at[pl.ds(s * RPT, RPT)]

        @pl.when(c == 0)
        def _():
            pltpu.sync_copy(stripe, agg0.at[pl.ds(s * RPT, RPT)])

        @pl.when(c == 1)
        def _():
            pltpu.sync_copy(stripe, agg1.at[pl.ds(s * RPT, RPT)])

    f = pl.kernel(
        body,
        out_type=[jax.ShapeDtypeStruct((NP, 128), jnp.float32),
                  jax.ShapeDtypeStruct((NP, 128), jnp.float32)],
        mesh=_mesh(),
        scratch_types=[
            pltpu.VMEM((2, BLK), jnp.int32),
            pltpu.VMEM((BLK, 128), jnp.float32),
            pltpu.VMEM((BLK, 128), jnp.float32),
            pltpu.VMEM((BLK, 128), jnp.float32),
            pltpu.VMEM_SHARED((NP, 128), jnp.float32),
        ],
    )
    return f(sdh, ns2, nr2, ew2)


def _tc_node_proj(x, wcat):
    BN = 1000

    def body(xr, wr, outr):
        outr[0] = jnp.dot(xr[...], wr[...], preferred_element_type=jnp.float32)

    return pl.pallas_call(
        body,
        grid=(N // BN, 6),
        in_specs=[pl.BlockSpec((BN, 128), lambda i, j: (i, 0)),
                  pl.BlockSpec((128, 128), lambda i, j: (0, j))],
        out_specs=pl.BlockSpec((1, BN, 128), lambda i, j: (j, i, 0)),
        out_shape=jax.ShapeDtypeStruct((6, N, 128), jnp.float32),
    )(x, wcat)


def _tc_edge_proj(edge_attr, we1, be1r):
    BE = 2000

    def body(ear, wr, br, outr):
        outr[0] = (jnp.dot(ear[...], wr[...], preferred_element_type=jnp.float32)
                   + br[0, 0:1])

    return pl.pallas_call(
        body,
        grid=(E // BE, 2),
        in_specs=[pl.BlockSpec((BE, 16), lambda i, j: (i, 0)),
                  pl.BlockSpec((16, 128), lambda i, j: (0, j)),
                  pl.BlockSpec((1, 8, 128), lambda i, j: (j, 0, 0))],
        out_specs=pl.BlockSpec((1, BE, 128), lambda i, j: (j, i, 0)),
        out_shape=jax.ShapeDtypeStruct((2, E, 128), jnp.float32),
    )(edge_attr, we1, be1r)


def _tc_node1(agg1h, cnt0, cnt1, p1, wi1, w2cat, bn1r, b3r):
    BN = 1000

    def body(aggr, c0r, c1r, p1r, wi1r, w2r, b1r, b3rr, outr):
        cnt = c0r[...][:, 0:1] + c1r[...][:, 0:1]
        dinv = 1.0 / jnp.maximum(cnt, 1.0)
        agg1 = jnp.concatenate([aggr[0], aggr[1]], axis=1) * dinv
        xn = jnp.concatenate([p1r[0], p1r[1]], axis=1)
        n1 = jnp.maximum(
            jnp.dot(agg1, wi1r[...], preferred_element_type=jnp.float32)
            + xn + b1r[...], 0.0)
        outr[...] = (jnp.dot(n1, w2r[...], preferred_element_type=jnp.float32)
                     + b3rr[...])

    return pl.pallas_call(
        body,
        grid=(N // BN,),
        in_specs=[pl.BlockSpec((2, BN, 128), lambda i: (0, i, 0)),
                  pl.BlockSpec((BN, 128), lambda i: (i, 0)),
                  pl.BlockSpec((BN, 128), lambda i: (i, 0)),
                  pl.BlockSpec((2, BN, 128), lambda i: (2, i, 0)),
                  pl.BlockSpec((256, 256), lambda i: (0, 0)),
                  pl.BlockSpec((256, 384), lambda i: (0, 0)),
                  pl.BlockSpec((1, 256), lambda i: (0, 0)),
                  pl.BlockSpec((1, 384), lambda i: (0, 0))],
        out_specs=pl.BlockSpec((BN, 384), lambda i: (i, 0)),
        out_shape=jax.ShapeDtypeStruct((N, 384), jnp.float32),
    )(agg1h, cnt0, cnt1, p1, wi1, w2cat, bn1r, b3r)


def _tc_edge2_mm(e1h, we2r, be2r):
    BE = 2000

    def body(er, wr, br, outr):
        outr[...] = (jnp.dot(er[0], wr[0], preferred_element_type=jnp.float32)
                     + jnp.dot(er[1], wr[1], preferred_element_type=jnp.float32)
                     + br[...])

    return pl.pallas_call(
        body,
        grid=(E // BE,),
        in_specs=[pl.BlockSpec((2, BE, 128), lambda i: (0, i, 0)),
                  pl.BlockSpec((2, 128, 128), lambda i: (0, 0, 0)),
                  pl.BlockSpec((1, 128), lambda i: (0, 0))],
        out_specs=pl.BlockSpec((BE, 128), lambda i: (i, 0)),
        out_shape=jax.ShapeDtypeStruct((E, 128), jnp.float32),
    )(e1h, we2r, be2r)


def _tc_final_partials(o2, agg2p0, agg2p1, cnt0, cnt1, wi2):
    BN = 1000
    NSTEP = N // BN

    def body(y2r, p0r, p1r, c0r, c1r, wi2r, outn, oute):
        agg2u = p0r[...] + p1r[...]
        oute[0] = jnp.broadcast_to(
            jnp.sum(agg2u, axis=0, keepdims=True), (8, 128))
        cnt = c0r[...][:, 0:1] + c1r[...][:, 0:1]
        dinv = 1.0 / jnp.maximum(cnt, 1.0)
        n2 = jnp.maximum(
            y2r[...] + jnp.dot(agg2u * dinv, wi2r[...],
                               preferred_element_type=jnp.float32), 0.0)
        outn[0] = jnp.broadcast_to(jnp.sum(n2, axis=0, keepdims=True), (8, 128))

    return pl.pallas_call(
        body,
        grid=(NSTEP,),
        in_specs=[pl.BlockSpec((BN, 128), lambda i: (i, 2)),
                  pl.BlockSpec((BN, 128), lambda i: (i, 0)),
                  pl.BlockSpec((BN, 128), lambda i: (i, 0)),
                  pl.BlockSpec((BN, 128), lambda i: (i, 0)),
                  pl.BlockSpec((BN, 128), lambda i: (i, 0)),
                  pl.BlockSpec((128, 128), lambda i: (0, 0))],
        out_specs=[pl.BlockSpec((1, 8, 128), lambda i: (i, 0, 0)),
                   pl.BlockSpec((1, 8, 128), lambda i: (i, 0, 0))],
        out_shape=[jax.ShapeDtypeStruct((NSTEP, 8, 128), jnp.float32),
                   jax.ShapeDtypeStruct((NSTEP, 8, 128), jnp.float32)],
    )(o2, agg2p0, agg2p1, cnt0, cnt1, wi2)


def _tc_readout(pn, pe, wgp):
    def body(pnr, per, wgpr, outr):
        sn = jnp.sum(pnr[...], axis=0, keepdims=True) / 8.0
        se = jnp.sum(per[...], axis=0, keepdims=True) / 8.0
        v = jnp.concatenate([sn / float(N), se / float(E)], axis=1)
        g = jnp.sum(v * wgpr[...][:, 0][None, :])
        outr[...] = jnp.full((8, 128), g, jnp.float32)

    nr = pn.shape[0] * pn.shape[1]
    pn2 = pn.reshape(nr, 128)
    pe2 = pe.reshape(nr, 128)
    return pl.pallas_call(
        body,
        in_specs=[pl.BlockSpec((nr, 128), lambda: (0, 0)),
                  pl.BlockSpec((nr, 128), lambda: (0, 0)),
                  pl.BlockSpec((256, 128), lambda: (0, 0))],
        out_specs=pl.BlockSpec((8, 128), lambda: (0, 0)),
        out_shape=jax.ShapeDtypeStruct((8, 128), jnp.float32),
    )(pn2, pe2, wgp)


def kernel(x, edge_index, edge_attr, We1, Ws1, Wr1, be1, Wn1, Wi1, bn1,
           We2, Ws2, Wr2, be2, Wn2, Wi2, bn2, Wgn, Wge, bg):
    ei = edge_index.astype(jnp.int32)
    src = ei[0]
    dst = ei[1]
    src2 = src.reshape(2, EH)
    dst2 = dst.reshape(2, EH)

    cnt0, cnt1 = _sc_counts(dst2)

    w1cat = jnp.concatenate([Ws1, Wr1, Wn1], axis=1)
    p1 = _tc_node_proj(x, w1cat)
    tsa, tsb, tra, trb = p1[0], p1[1], p1[2], p1[3]
    be1r = jnp.broadcast_to(be1.reshape(2, 1, 128), (2, 8, 128))
    ea2 = _tc_edge_proj(edge_attr, We1, be1r)

    e1h, agg1h = _sc_pass1(ei, tsa, tsb, tra, trb, ea2)

    w2cat = jnp.concatenate([Ws2, Wr2, Wn2], axis=1)
    b3r = jnp.concatenate([jnp.zeros((256,), jnp.float32), bn2]).reshape(1, 384)
    o2 = _tc_node1(agg1h, cnt0, cnt1, p1, Wi1, w2cat,
                   bn1.reshape(1, 256), b3r)
    ns2 = o2[:, 0:128]
    nr2 = o2[:, 128:256]

    ew = _tc_edge2_mm(e1h, We2.reshape(2, 128, 128), be2.reshape(1, 128))
    ew2 = ew.reshape(2, EH, 128)

    sdh = jnp.stack([src2, dst2], axis=1)
    agg2p0, agg2p1 = _sc_pass2(sdh, ns2, nr2, ew2)

    wgp = jnp.pad(jnp.concatenate([Wgn, Wge], axis=0), ((0, 0), (0, 127)))
    pn, pe = _tc_final_partials(o2, agg2p0, agg2p1, cnt0, cnt1, Wi2)
    out = _tc_readout(pn, pe, wgp)
    return out[0, 0:1] + bg
